# Initial kernel scaffold; baseline (speedup 1.0000x reference)
import jax
import jax.numpy as jnp
from jax.experimental import pallas as pl


def kernel(x, attention_mask, gate_w, expert_w, expert_b):
    raise NotImplementedError("write your pallas kernel here")


# speedup vs baseline: 3.7134x; 3.7134x over previous
import jax
import jax.numpy as jnp
from jax.experimental import pallas as pl
from jax.experimental.pallas import tpu as pltpu

NE = 8
NB = 2
D = 2048
B = 16
T = 32
S = NB * B
TJ = 256
NJ = D // TJ


def _gate_kernel(x_ref, gw_ref, topv_ref, topi_ref):
    x = x_ref[...]
    xavg = jnp.sum(x, axis=1) * (1.0 / T)
    logits = jax.lax.dot_general(
        xavg, gw_ref[...],
        dimension_numbers=(((1,), (1,)), ((), ())),
        preferred_element_type=jnp.float32)
    m = jnp.max(logits, axis=1, keepdims=True)
    ex = jnp.exp(logits - m)
    p = ex / jnp.sum(ex, axis=1, keepdims=True)
    iota = jax.lax.broadcasted_iota(jnp.int32, (B, NE), 1)
    v0 = jnp.max(p, axis=1, keepdims=True)
    i0 = jnp.min(jnp.where(p == v0, iota, NE), axis=1, keepdims=True)
    p1 = jnp.where(iota == i0, -1.0, p)
    v1 = jnp.max(p1, axis=1, keepdims=True)
    i1 = jnp.min(jnp.where(p1 == v1, iota, NE), axis=1, keepdims=True)
    topv_ref[...] = jnp.concatenate([v0, v1], axis=1)
    topi_ref[...] = jnp.concatenate([i0, i1], axis=1)


def _ffn_kernel(eid_ref, order_ref, x_ref, w_ref, b_ref, out_ref):
    i = pl.program_id(1)
    s = order_ref[i]
    bidx = s // NB
    xb = x_ref[pl.ds(bidx * T, T), :]
    acc = jax.lax.dot_general(
        xb, w_ref[0],
        dimension_numbers=(((1,), (1,)), ((), ())),
        preferred_element_type=jnp.float32)
    out_ref[...] = acc + b_ref[0]


def kernel(x, attention_mask, gate_w, expert_w, expert_b):
    topv, topi = pl.pallas_call(
        _gate_kernel,
        out_shape=[
            jax.ShapeDtypeStruct((B, NB), jnp.float32),
            jax.ShapeDtypeStruct((B, NB), jnp.int32),
        ],
    )(x, gate_w)

    beam_scores = topv.reshape(S)
    es = topi.reshape(S)
    expert_route = es[:, None]

    order = jnp.argsort(es, stable=True).astype(jnp.int32)
    eid_sorted = es[order]

    xf = x.reshape(B * T, D)
    grid_spec = pltpu.PrefetchScalarGridSpec(
        num_scalar_prefetch=2,
        grid=(NJ, S),
        in_specs=[
            pl.BlockSpec((B * T, D), lambda j, i, eid, order: (0, 0)),
            pl.BlockSpec((1, TJ, D), lambda j, i, eid, order: (eid[i], j, 0)),
            pl.BlockSpec((1, 1, TJ), lambda j, i, eid, order: (eid[i], 0, j)),
        ],
        out_specs=pl.BlockSpec((T, TJ), lambda j, i, eid, order: (order[i], j)),
    )
    out_flat = pl.pallas_call(
        _ffn_kernel,
        grid_spec=grid_spec,
        out_shape=jax.ShapeDtypeStruct((S * T, D), jnp.float32),
        compiler_params=pltpu.CompilerParams(
            dimension_semantics=("arbitrary", "arbitrary")),
    )(eid_sorted, order, xf, expert_w, expert_b.reshape(NE, 1, D))

    candidate_output = out_flat.reshape(S, T, D)
    return candidate_output, beam_scores, expert_route


# speedup vs baseline: 4.8810x; 1.3144x over previous
import jax
import jax.numpy as jnp
from jax.experimental import pallas as pl
from jax.experimental.pallas import tpu as pltpu

NE = 8
NB = 2
D = 2048
B = 16
T = 32
S = NB * B
TJ = 512
NJ = D // TJ


def _gate_kernel(x_ref, gw_ref, topv_ref, topi_ref):
    x = x_ref[...]
    xavg = jnp.sum(x, axis=1) * (1.0 / T)
    logits = jax.lax.dot_general(
        xavg, gw_ref[...],
        dimension_numbers=(((1,), (1,)), ((), ())),
        preferred_element_type=jnp.float32)
    m = jnp.max(logits, axis=1, keepdims=True)
    ex = jnp.exp(logits - m)
    p = ex / jnp.sum(ex, axis=1, keepdims=True)
    iota = jax.lax.broadcasted_iota(jnp.int32, (B, NE), 1)
    v0 = jnp.max(p, axis=1, keepdims=True)
    i0 = jnp.min(jnp.where(p == v0, iota, NE), axis=1, keepdims=True)
    p1 = jnp.where(iota == i0, -1.0, p)
    v1 = jnp.max(p1, axis=1, keepdims=True)
    i1 = jnp.min(jnp.where(p1 == v1, iota, NE), axis=1, keepdims=True)
    topv_ref[...] = jnp.concatenate([v0, v1], axis=1)
    topi_ref[...] = jnp.concatenate([i0, i1], axis=1)


def _ffn_kernel(eid_ref, order_ref, x_ref, w_ref, b_ref, out_ref):
    i = pl.program_id(1)
    s = order_ref[i]
    bidx = s // NB
    xb = x_ref[pl.ds(bidx * T, T), :]
    acc = jax.lax.dot_general(
        xb, w_ref[0].astype(jnp.bfloat16),
        dimension_numbers=(((1,), (1,)), ((), ())),
        preferred_element_type=jnp.float32)
    out_ref[...] = acc + b_ref[0]


def kernel(x, attention_mask, gate_w, expert_w, expert_b):
    topv, topi = pl.pallas_call(
        _gate_kernel,
        out_shape=[
            jax.ShapeDtypeStruct((B, NB), jnp.float32),
            jax.ShapeDtypeStruct((B, NB), jnp.int32),
        ],
    )(x, gate_w)

    beam_scores = topv.reshape(S)
    es = topi.reshape(S)
    expert_route = es[:, None]

    order = jnp.argsort(es, stable=True).astype(jnp.int32)
    eid_sorted = es[order]

    xf = x.reshape(B * T, D).astype(jnp.bfloat16)
    grid_spec = pltpu.PrefetchScalarGridSpec(
        num_scalar_prefetch=2,
        grid=(NJ, S),
        in_specs=[
            pl.BlockSpec((B * T, D), lambda j, i, eid, order: (0, 0)),
            pl.BlockSpec((1, TJ, D), lambda j, i, eid, order: (eid[i], j, 0)),
            pl.BlockSpec((1, 1, TJ), lambda j, i, eid, order: (eid[i], 0, j)),
        ],
        out_specs=pl.BlockSpec((T, TJ), lambda j, i, eid, order: (order[i], j)),
    )
    out_flat = pl.pallas_call(
        _ffn_kernel,
        grid_spec=grid_spec,
        out_shape=jax.ShapeDtypeStruct((S * T, D), jnp.float32),
        compiler_params=pltpu.CompilerParams(
            dimension_semantics=("arbitrary", "arbitrary")),
    )(eid_sorted, order, xf, expert_w, expert_b.reshape(NE, 1, D))

    candidate_output = out_flat.reshape(S, T, D)
    return candidate_output, beam_scores, expert_route


# speedup vs baseline: 5.6838x; 1.1645x over previous
import jax
import jax.numpy as jnp
from jax.experimental import pallas as pl
from jax.experimental.pallas import tpu as pltpu

NE = 8
NB = 2
D = 2048
B = 16
T = 32
S = NB * B
TJ = 1024
NJ = D // TJ


def _gate_kernel(x_ref, gw_ref, topv_ref, topi_ref):
    x = x_ref[...]
    xavg = jnp.sum(x, axis=1) * (1.0 / T)
    logits = jax.lax.dot_general(
        xavg, gw_ref[...],
        dimension_numbers=(((1,), (1,)), ((), ())),
        preferred_element_type=jnp.float32)
    m = jnp.max(logits, axis=1, keepdims=True)
    ex = jnp.exp(logits - m)
    p = ex / jnp.sum(ex, axis=1, keepdims=True)
    iota = jax.lax.broadcasted_iota(jnp.int32, (B, NE), 1)
    v0 = jnp.max(p, axis=1, keepdims=True)
    i0 = jnp.min(jnp.where(p == v0, iota, NE), axis=1, keepdims=True)
    p1 = jnp.where(iota == i0, -1.0, p)
    v1 = jnp.max(p1, axis=1, keepdims=True)
    i1 = jnp.min(jnp.where(p1 == v1, iota, NE), axis=1, keepdims=True)
    topv_ref[...] = jnp.concatenate([v0, v1], axis=1)
    topi_ref[...] = jnp.concatenate([i0, i1], axis=1)


def _ffn_kernel(eid_ref, order_ref, x_ref, w_ref, b_ref, out_ref):
    i = pl.program_id(1)
    s = order_ref[i]
    bidx = s // NB
    xb = x_ref[pl.ds(bidx * T, T), :]
    acc = jax.lax.dot_general(
        xb, w_ref[0].astype(jnp.bfloat16),
        dimension_numbers=(((1,), (1,)), ((), ())),
        preferred_element_type=jnp.float32)
    out_ref[...] = acc + b_ref[0]


def kernel(x, attention_mask, gate_w, expert_w, expert_b):
    topv, topi = pl.pallas_call(
        _gate_kernel,
        out_shape=[
            jax.ShapeDtypeStruct((B, NB), jnp.float32),
            jax.ShapeDtypeStruct((B, NB), jnp.int32),
        ],
    )(x, gate_w)

    beam_scores = topv.reshape(S)
    es = topi.reshape(S)
    expert_route = es[:, None]

    order = jnp.argsort(es, stable=True).astype(jnp.int32)
    eid_sorted = es[order]

    xf = x.reshape(B * T, D).astype(jnp.bfloat16)
    grid_spec = pltpu.PrefetchScalarGridSpec(
        num_scalar_prefetch=2,
        grid=(NJ, S),
        in_specs=[
            pl.BlockSpec((B * T, D), lambda j, i, eid, order: (0, 0)),
            pl.BlockSpec((1, TJ, D), lambda j, i, eid, order: (eid[i], j, 0)),
            pl.BlockSpec((1, 1, TJ), lambda j, i, eid, order: (eid[i], 0, j)),
        ],
        out_specs=pl.BlockSpec((T, TJ), lambda j, i, eid, order: (order[i], j)),
    )
    out_flat = pl.pallas_call(
        _ffn_kernel,
        grid_spec=grid_spec,
        out_shape=jax.ShapeDtypeStruct((S * T, D), jnp.float32),
        compiler_params=pltpu.CompilerParams(
            dimension_semantics=("arbitrary", "arbitrary")),
    )(eid_sorted, order, xf, expert_w, expert_b.reshape(NE, 1, D))

    candidate_output = out_flat.reshape(S, T, D)
    return candidate_output, beam_scores, expert_route


# speedup vs baseline: 6.1890x; 1.0889x over previous
import jax
import jax.numpy as jnp
from jax.experimental import pallas as pl
from jax.experimental.pallas import tpu as pltpu

NE = 8
NB = 2
D = 2048
B = 16
T = 32
S = NB * B
TJ = 2048
NJ = D // TJ


def _gate_kernel(x_ref, gw_ref, topv_ref, topi_ref):
    x = x_ref[...]
    xavg = jnp.sum(x, axis=1) * (1.0 / T)
    logits = jax.lax.dot_general(
        xavg, gw_ref[...],
        dimension_numbers=(((1,), (1,)), ((), ())),
        preferred_element_type=jnp.float32)
    m = jnp.max(logits, axis=1, keepdims=True)
    ex = jnp.exp(logits - m)
    p = ex / jnp.sum(ex, axis=1, keepdims=True)
    iota = jax.lax.broadcasted_iota(jnp.int32, (B, NE), 1)
    v0 = jnp.max(p, axis=1, keepdims=True)
    i0 = jnp.min(jnp.where(p == v0, iota, NE), axis=1, keepdims=True)
    p1 = jnp.where(iota == i0, -1.0, p)
    v1 = jnp.max(p1, axis=1, keepdims=True)
    i1 = jnp.min(jnp.where(p1 == v1, iota, NE), axis=1, keepdims=True)
    topv_ref[...] = jnp.concatenate([v0, v1], axis=1)
    topi_ref[...] = jnp.concatenate([i0, i1], axis=1)


def _ffn_kernel(eid_ref, order_ref, x_ref, w_ref, b_ref, out_ref):
    i = pl.program_id(1)
    s = order_ref[i]
    bidx = s // NB
    xb = x_ref[pl.ds(bidx * T, T), :]
    acc = jax.lax.dot_general(
        xb, w_ref[0].astype(jnp.bfloat16),
        dimension_numbers=(((1,), (1,)), ((), ())),
        preferred_element_type=jnp.float32)
    out_ref[...] = acc + b_ref[0]


def kernel(x, attention_mask, gate_w, expert_w, expert_b):
    topv, topi = pl.pallas_call(
        _gate_kernel,
        out_shape=[
            jax.ShapeDtypeStruct((B, NB), jnp.float32),
            jax.ShapeDtypeStruct((B, NB), jnp.int32),
        ],
    )(x, gate_w)

    beam_scores = topv.reshape(S)
    es = topi.reshape(S)
    expert_route = es[:, None]

    order = jnp.argsort(es, stable=True).astype(jnp.int32)
    eid_sorted = es[order]

    xf = x.reshape(B * T, D).astype(jnp.bfloat16)
    grid_spec = pltpu.PrefetchScalarGridSpec(
        num_scalar_prefetch=2,
        grid=(NJ, S),
        in_specs=[
            pl.BlockSpec((B * T, D), lambda j, i, eid, order: (0, 0)),
            pl.BlockSpec((1, TJ, D), lambda j, i, eid, order: (eid[i], j, 0)),
            pl.BlockSpec((1, 1, TJ), lambda j, i, eid, order: (eid[i], 0, j)),
        ],
        out_specs=pl.BlockSpec((T, TJ), lambda j, i, eid, order: (order[i], j)),
    )
    out_flat = pl.pallas_call(
        _ffn_kernel,
        grid_spec=grid_spec,
        out_shape=jax.ShapeDtypeStruct((S * T, D), jnp.float32),
        compiler_params=pltpu.CompilerParams(
            dimension_semantics=("arbitrary", "arbitrary")),
    )(eid_sorted, order, xf, expert_w, expert_b.reshape(NE, 1, D))

    candidate_output = out_flat.reshape(S, T, D)
    return candidate_output, beam_scores, expert_route


# speedup vs baseline: 10.9052x; 1.7620x over previous
import jax
import jax.numpy as jnp
from jax.experimental import pallas as pl
from jax.experimental.pallas import tpu as pltpu

NE = 8
NB = 2
D = 2048
B = 16
T = 32
S = NB * B
TJ = 1024
NJ = D // TJ


def _gate_kernel(x_ref, gw_ref, topv_ref, topi_ref, er0_ref, er1_ref):
    x = x_ref[...]
    xavg = jnp.sum(x, axis=1) * (1.0 / T)
    logits = jax.lax.dot_general(
        xavg, gw_ref[...],
        dimension_numbers=(((1,), (1,)), ((), ())),
        preferred_element_type=jnp.float32)
    m = jnp.max(logits, axis=1, keepdims=True)
    ex = jnp.exp(logits - m)
    p = ex / jnp.sum(ex, axis=1, keepdims=True)
    iota = jax.lax.broadcasted_iota(jnp.int32, (B, NE), 1)
    v0 = jnp.max(p, axis=1, keepdims=True)
    i0 = jnp.min(jnp.where(p == v0, iota, NE), axis=1, keepdims=True)
    p1 = jnp.where(iota == i0, -1.0, p)
    v1 = jnp.max(p1, axis=1, keepdims=True)
    i1 = jnp.min(jnp.where(p1 == v1, iota, NE), axis=1, keepdims=True)
    topv_ref[...] = jnp.concatenate([v0, v1], axis=1)
    topi_ref[...] = jnp.concatenate([i0, i1], axis=1)
    er0_ref[...] = jnp.broadcast_to(i0[:, None, :], (B, T, 1))
    er1_ref[...] = jnp.broadcast_to(i1[:, None, :], (B, T, 1))


def _ffn_kernel(x_ref, w_ref, b_ref, er0_ref, er1_ref, out_ref):
    e = pl.program_id(1)
    t = jax.lax.dot_general(
        x_ref[...], w_ref[0].astype(jnp.bfloat16),
        dimension_numbers=(((1,), (1,)), ((), ())),
        preferred_element_type=jnp.float32)
    t3 = t.reshape(B, T, TJ) + b_ref[0][None]
    c0 = (er0_ref[...] == e).astype(jnp.float32) * t3
    c1 = (er1_ref[...] == e).astype(jnp.float32) * t3

    @pl.when(e == 0)
    def _():
        out_ref[:, :T, :] = c0
        out_ref[:, T:, :] = c1

    @pl.when(e != 0)
    def _():
        out_ref[:, :T, :] += c0
        out_ref[:, T:, :] += c1


def kernel(x, attention_mask, gate_w, expert_w, expert_b):
    topv, topi, er0, er1 = pl.pallas_call(
        _gate_kernel,
        out_shape=[
            jax.ShapeDtypeStruct((B, NB), jnp.float32),
            jax.ShapeDtypeStruct((B, NB), jnp.int32),
            jax.ShapeDtypeStruct((B, T, 1), jnp.int32),
            jax.ShapeDtypeStruct((B, T, 1), jnp.int32),
        ],
    )(x, gate_w)

    beam_scores = topv.reshape(S)
    expert_route = topi.reshape(S)[:, None]

    xf = x.reshape(B * T, D).astype(jnp.bfloat16)
    out = pl.pallas_call(
        _ffn_kernel,
        grid=(NJ, NE),
        in_specs=[
            pl.BlockSpec((B * T, D), lambda j, e: (0, 0)),
            pl.BlockSpec((1, TJ, D), lambda j, e: (e, j, 0)),
            pl.BlockSpec((1, 1, TJ), lambda j, e: (e, 0, j)),
            pl.BlockSpec((B, T, 1), lambda j, e: (0, 0, 0)),
            pl.BlockSpec((B, T, 1), lambda j, e: (0, 0, 0)),
        ],
        out_specs=pl.BlockSpec((B, NB * T, TJ), lambda j, e: (0, 0, j)),
        out_shape=jax.ShapeDtypeStruct((B, NB * T, D), jnp.float32),
        compiler_params=pltpu.CompilerParams(
            dimension_semantics=("arbitrary", "arbitrary")),
    )(xf, expert_w, expert_b.reshape(NE, 1, D), er0, er1)

    candidate_output = out.reshape(S, T, D)
    return candidate_output, beam_scores, expert_route
